# baseline (speedup 1.0000x reference)
import functools

import jax
import jax.numpy as jnp
from jax import lax
from jax.experimental import pallas as pl
from jax.experimental.pallas import tpu as pltpu
from jax.experimental.pallas import tpu_sc as plsc

NC = 2
NS = 16
NW = NC * NS
LANES = 16


def _make_kernel(B, L, V, D, C):
    N = B * L
    per_w = N // NW
    BLK = 128
    nblk = per_w // BLK
    assert per_w % BLK == 0 and N % NW == 0 and per_w % C == 0

    mesh = plsc.VectorSubcoreMesh(core_axis_name="c", subcore_axis_name="s")

    @functools.partial(
        pl.kernel,
        out_type=jax.ShapeDtypeStruct((N, D), jnp.float32),
        mesh=mesh,
        compiler_params=pltpu.CompilerParams(use_tc_tiling_on_sc=False),
        scratch_types=[
            pltpu.VMEM((nblk, BLK), jnp.int32),
            pltpu.VMEM((BLK, D), jnp.float32),
            pltpu.VMEM((2 * C, D), jnp.float32),
            pltpu.SemaphoreType.DMA,
        ],
    )
    def k(x_hbm, tok_hbm, pos_hbm, out_hbm, idx_v, rows_v, pos_v, gsem):
        wid = lax.axis_index("s") * NC + lax.axis_index("c")
        base = wid * per_w

        pltpu.sync_copy(x_hbm.at[wid], idx_v)
        pltpu.sync_copy(pos_hbm, pos_v.at[pl.ds(0, C)])
        pltpu.sync_copy(pos_hbm, pos_v.at[pl.ds(C, C)])

        def block(kb, carry):
            pltpu.async_copy(tok_hbm.at[idx_v.at[kb]], rows_v, gsem).wait()
            pbase = lax.rem(kb * BLK, C)

            def addr(r, c2):
                for c in range(D // LANES):
                    sl = pl.ds(c * LANES, LANES)
                    rows_v[r, sl] = rows_v[r, sl] + pos_v[pbase + r, sl]
                return c2

            lax.fori_loop(0, BLK, addr, 0)
            pltpu.sync_copy(rows_v, out_hbm.at[pl.ds(base + kb * BLK, BLK)])
            return carry

        lax.fori_loop(0, nblk, block, 0)

    return k


def kernel(x, tok_table, pos_table):
    B, L = x.shape
    V, D = tok_table.shape
    C = pos_table.shape[0]
    N = B * L
    per_w = N // NW
    BLK = 128
    x3 = x.reshape(NW, per_w // BLK, BLK)
    k = _make_kernel(B, L, V, D, C)
    out = k(x3, tok_table, pos_table)
    return out.reshape(B, L, D)


# speedup vs baseline: 1.4942x; 1.4942x over previous
import functools

import jax
import jax.numpy as jnp
from jax import lax
from jax.experimental import pallas as pl
from jax.experimental.pallas import tpu as pltpu
from jax.experimental.pallas import tpu_sc as plsc

NC = 2
NS = 16
NW = NC * NS
LANES = 16
NBUF = 4
PREF = 2


def _make_kernel(B, L, V, D, C):
    N = B * L
    per_w = N // NW
    BLK = 128
    nblk = per_w // BLK
    assert per_w % BLK == 0 and N % NW == 0 and per_w % C == 0
    assert nblk % NBUF == 0

    mesh = plsc.VectorSubcoreMesh(core_axis_name="c", subcore_axis_name="s")

    @functools.partial(
        pl.kernel,
        out_type=jax.ShapeDtypeStruct((N, D), jnp.float32),
        mesh=mesh,
        compiler_params=pltpu.CompilerParams(use_tc_tiling_on_sc=False),
        scratch_types=[
            pltpu.VMEM((nblk, BLK), jnp.int32),
            pltpu.VMEM((NBUF, BLK, D), jnp.float32),
            pltpu.VMEM((2 * C, D), jnp.float32),
            [pltpu.SemaphoreType.DMA] * NBUF,
            [pltpu.SemaphoreType.DMA] * NBUF,
        ],
    )
    def k(x_hbm, tok_hbm, pos_hbm, out_hbm, idx_v, rows_v, pos_v, gsems, wsems):
        wid = lax.axis_index("s") * NC + lax.axis_index("c")
        base = wid * per_w

        pltpu.sync_copy(x_hbm.at[wid], idx_v)
        pltpu.sync_copy(pos_hbm, pos_v.at[pl.ds(0, C)])
        pltpu.sync_copy(pos_hbm, pos_v.at[pl.ds(C, C)])

        def gather(kb, p):
            return pltpu.make_async_copy(
                tok_hbm.at[idx_v.at[kb]], rows_v.at[p], gsems[p])

        def writeback(kb, p):
            return pltpu.make_async_copy(
                rows_v.at[p], out_hbm.at[pl.ds(base + kb * BLK, BLK)], wsems[p])

        for j in range(PREF):
            gather(j, j).start()

        def step(k2, j, carry):
            kb = NBUF * k2 + j
            p = j
            q = (j + PREF) % NBUF

            gather(kb, p).wait()

            pbase = lax.rem(kb * BLK, C)

            @plsc.parallel_loop(0, BLK, step=1, unroll=8)
            def addr(r):
                for c in range(D // LANES):
                    sl = pl.ds(c * LANES, LANES)
                    rows_v[p, r, sl] = rows_v[p, r, sl] + pos_v[pbase + r, sl]

            writeback(kb, p).start()

            if j < PREF:
                @pl.when(k2 >= 1)
                def _():
                    writeback(kb - PREF, q).wait()

                gather(kb + PREF, q).start()
            else:
                writeback(kb - PREF, q).wait()

                @pl.when(k2 < nblk // NBUF - 1)
                def _():
                    gather(kb + PREF, q).start()
            return carry

        def round4(k2, carry):
            for j in range(NBUF):
                carry = step(k2, j, carry)
            return carry

        lax.fori_loop(0, nblk // NBUF, round4, 0)

        for kb in range(nblk - PREF, nblk):
            writeback(kb, kb % NBUF).wait()

    return k


def kernel(x, tok_table, pos_table):
    B, L = x.shape
    V, D = tok_table.shape
    C = pos_table.shape[0]
    N = B * L
    per_w = N // NW
    BLK = 128
    x3 = x.reshape(NW, per_w // BLK, BLK)
    k = _make_kernel(B, L, V, D, C)
    out = k(x3, tok_table, pos_table)
    return out.reshape(B, L, D)
